# Initial kernel scaffold; baseline (speedup 1.0000x reference)
import functools

import jax
import jax.numpy as jnp
from jax import lax
from jax.experimental import pallas as pl
from jax.experimental.pallas import tpu as pltpu
from jax.experimental.pallas import tpu_sc as plsc

N_NODES = 10000
N_EDGES = 320000
HID = 128
NF = 128
NBLK = 2
RAD = 50
NGRAPH = 64
FILM = 16
CUT = 10.0

NC = 2
NS = 16
NW = NC * NS
EPW = N_EDGES // NW
CH = 80
NCH = EPW // CH
RPT = N_NODES // NS

_f32 = jnp.float32


def _silu(x):
    return x * jax.nn.sigmoid(x)


def _sc_gather_body(hr_hbm, hc_hbm, row_hbm, col_hbm, hrg_hbm, hcg_hbm,
                    idxr, idxc, buf_a, buf_b, sem_a, sem_b):
    cid = lax.axis_index("c")
    sid = lax.axis_index("s")
    wid = sid * NC + cid
    base0 = wid * EPW

    def chunk(j, carry):
        base = base0 + j * CH
        pltpu.sync_copy(row_hbm.at[pl.ds(base, CH)], idxr)
        pltpu.sync_copy(col_hbm.at[pl.ds(base, CH)], idxc)
        cp_a = pltpu.async_copy(hr_hbm.at[idxr], buf_a, sem_a)
        cp_b = pltpu.async_copy(hc_hbm.at[idxc], buf_b, sem_b)
        cp_a.wait()
        cp_b.wait()
        pltpu.sync_copy(buf_a, hrg_hbm.at[pl.ds(base, CH)])
        pltpu.sync_copy(buf_b, hcg_hbm.at[pl.ds(base, CH)])
        return carry

    lax.fori_loop(0, NCH, chunk, 0, unroll=False)


_sc_gather = pl.kernel(
    _sc_gather_body,
    out_type=(
        jax.ShapeDtypeStruct((N_EDGES, HID), _f32),
        jax.ShapeDtypeStruct((N_EDGES, HID), _f32),
    ),
    mesh=plsc.VectorSubcoreMesh(core_axis_name="c", subcore_axis_name="s"),
    scratch_types=[
        pltpu.VMEM((CH,), jnp.int32),
        pltpu.VMEM((CH,), jnp.int32),
        pltpu.VMEM((CH, HID), _f32),
        pltpu.VMEM((CH, HID), _f32),
        pltpu.SemaphoreType.DMA,
        pltpu.SemaphoreType.DMA,
    ],
)


def _sc_scatter_body(e_hbm, col_hbm, zeros_hbm, agg_hbm,
                     idxc, dbuf, acc, sem):
    del sem
    cid = lax.axis_index("c")
    sid = lax.axis_index("s")
    wid = sid * NC + cid
    base0 = wid * EPW

    pltpu.sync_copy(zeros_hbm.at[pl.ds(sid * RPT, RPT)],
                    acc.at[pl.ds(sid * RPT, RPT)])
    plsc.subcore_barrier()

    def chunk(j, carry):
        base = base0 + j * CH
        pltpu.sync_copy(col_hbm.at[pl.ds(base, CH)], idxc.at[0])
        pltpu.sync_copy(e_hbm.at[pl.ds(base, CH)], dbuf)
        pltpu.sync_copy(dbuf, acc.at[idxc.at[0]], add=True)
        return carry

    lax.fori_loop(0, NCH, chunk, 0, unroll=False)
    plsc.subcore_barrier()
    pltpu.sync_copy(acc.at[pl.ds(sid * RPT, RPT)],
                    agg_hbm.at[cid, pl.ds(sid * RPT, RPT)])


_sc_scatter = pl.kernel(
    _sc_scatter_body,
    out_type=jax.ShapeDtypeStruct((NC, N_NODES, HID), _f32),
    mesh=plsc.VectorSubcoreMesh(core_axis_name="c", subcore_axis_name="s"),
    scratch_types=[
        pltpu.VMEM((1, CH), jnp.int32),
        pltpu.VMEM((CH, HID), _f32),
        pltpu.VMEM_SHARED((N_NODES, HID), _f32),
        pltpu.SemaphoreType.DMA,
    ],
)


TN = 2000
TE = 2000


def _prep_body(an_ref, emb_ref, a_ref, b_ref, x_ref, hr_ref, hc_ref):
    an = an_ref[:]
    ids = lax.broadcasted_iota(jnp.int32, (1, 100), 1)
    oh = (an == ids).astype(_f32)
    x = jnp.dot(oh, emb_ref[:], preferred_element_type=_f32)
    x_ref[:] = x
    hr_ref[:] = jnp.dot(x, a_ref[:], preferred_element_type=_f32)
    hc_ref[:] = jnp.dot(x, b_ref[:], preferred_element_type=_f32)


def _prep_call(an2, atom_emb, a_w, b_w):
    grid = N_NODES // TN
    return pl.pallas_call(
        _prep_body,
        grid=(grid,),
        in_specs=[
            pl.BlockSpec((TN, 1), lambda i: (i, 0)),
            pl.BlockSpec((100, HID), lambda i: (0, 0)),
            pl.BlockSpec((HID, HID), lambda i: (0, 0)),
            pl.BlockSpec((HID, HID), lambda i: (0, 0)),
        ],
        out_specs=[
            pl.BlockSpec((TN, HID), lambda i: (i, 0)),
            pl.BlockSpec((TN, HID), lambda i: (i, 0)),
            pl.BlockSpec((TN, HID), lambda i: (i, 0)),
        ],
        out_shape=[
            jax.ShapeDtypeStruct((N_NODES, HID), _f32),
            jax.ShapeDtypeStruct((N_NODES, HID), _f32),
            jax.ShapeDtypeStruct((N_NODES, HID), _f32),
        ],
    )(an2, atom_emb, a_w, b_w)


def _edge_body(d_ref, hrg_ref, hcg_ref, c_ref, b1_ref, w2_ref, b2_ref,
               w3_ref, b3_ref, out_ref):
    step = CUT / (RAD - 1)
    coeff = -0.5 / (step * step)
    off = lax.broadcasted_iota(_f32, (1, RAD), 1) * step
    d = d_ref[:]
    diff = d - off
    rbf = jnp.exp(coeff * diff * diff)
    pre = (hrg_ref[:] + hcg_ref[:] + b1_ref[:]
           + jnp.dot(rbf, c_ref[:], preferred_element_type=_f32))
    e1 = _silu(pre)
    z = jnp.dot(e1, w2_ref[:], preferred_element_type=_f32) + b2_ref[:]
    e2 = _silu(z)
    out_ref[:] = jnp.dot(e2, w3_ref[:], preferred_element_type=_f32) + b3_ref[:]


def _edge_call(d2, hrg, hcg, c_w, b1, w2, b2, w3, b3):
    grid = N_EDGES // TE
    return pl.pallas_call(
        _edge_body,
        grid=(grid,),
        in_specs=[
            pl.BlockSpec((TE, 1), lambda i: (i, 0)),
            pl.BlockSpec((TE, HID), lambda i: (i, 0)),
            pl.BlockSpec((TE, HID), lambda i: (i, 0)),
            pl.BlockSpec((RAD, NF), lambda i: (0, 0)),
            pl.BlockSpec((1, NF), lambda i: (0, 0)),
            pl.BlockSpec((NF, NF), lambda i: (0, 0)),
            pl.BlockSpec((1, NF), lambda i: (0, 0)),
            pl.BlockSpec((NF, HID), lambda i: (0, 0)),
            pl.BlockSpec((1, HID), lambda i: (0, 0)),
        ],
        out_specs=pl.BlockSpec((TE, HID), lambda i: (i, 0)),
        out_shape=jax.ShapeDtypeStruct((N_EDGES, HID), _f32),
    )(d2, hrg, hcg, c_w, b1, w2, b2, w3, b3)


def _node_body(emit_proj, x_ref, a0_ref, a1_ref, up_ref, uq_ref, ub1_ref,
               uw2_ref, ub2_ref, iw1_ref, ib1_ref, iw2_ref, ib2_ref,
               g_ref, b_ref, an_ref, bn_ref, *out_refs):
    x = x_ref[:]
    agg = a0_ref[:] + a1_ref[:]
    u = (jnp.dot(x, up_ref[:], preferred_element_type=_f32)
         + jnp.dot(agg, uq_ref[:], preferred_element_type=_f32) + ub1_ref[:])
    u = jnp.dot(_silu(u), uw2_ref[:], preferred_element_type=_f32) + ub2_ref[:]
    t = _silu(jnp.dot(u, iw1_ref[:], preferred_element_type=_f32) + ib1_ref[:])
    v = jnp.dot(t, iw2_ref[:], preferred_element_type=_f32) + ib2_ref[:]
    y = x + v
    m = jnp.mean(y, axis=-1, keepdims=True)
    yc = y - m
    var = jnp.mean(yc * yc, axis=-1, keepdims=True)
    xn = yc / jnp.sqrt(var + 1e-5) * g_ref[:] + b_ref[:]
    out_refs[0][:] = xn
    if emit_proj:
        out_refs[1][:] = jnp.dot(xn, an_ref[:], preferred_element_type=_f32)
        out_refs[2][:] = jnp.dot(xn, bn_ref[:], preferred_element_type=_f32)


def _node_call(emit_proj, x, a0, a1, up, uq, ub1, uw2, ub2, iw1, ib1, iw2,
               ib2, g, b, an_w, bn_w):
    grid = N_NODES // TN
    n_out = 3 if emit_proj else 1
    full = lambda shape: pl.BlockSpec(shape, lambda i: (0, 0))
    rows = pl.BlockSpec((TN, HID), lambda i: (i, 0))
    out = pl.pallas_call(
        functools.partial(_node_body, emit_proj),
        grid=(grid,),
        in_specs=[rows, rows, rows,
                  full((HID, HID)), full((HID, HID)), full((1, HID)),
                  full((HID, HID)), full((1, HID)),
                  full((HID, HID)), full((1, HID)),
                  full((HID, HID)), full((1, HID)),
                  full((1, HID)), full((1, HID)),
                  full((HID, HID)), full((HID, HID))],
        out_specs=[rows] * n_out,
        out_shape=[jax.ShapeDtypeStruct((N_NODES, HID), _f32)] * n_out,
    )(x, a0, a1, up, uq, ub1, uw2, ub2, iw1, ib1, iw2, ib2, g, b, an_w, bn_w)
    return out


def _readout_body(x_ref, batch_ref, dom_ref, dtab_ref, fgw_ref, fgb_ref,
                  fbw_ref, fbb_ref, hw1_ref, hb1_ref, hw2_ref, hb2_ref,
                  hw3_ref, hb3_ref, out_ref):
    bvec = batch_ref[:]
    gids = lax.broadcasted_iota(jnp.int32, (1, NGRAPH), 1)
    oh = (bvec == gids).astype(_f32)
    gf = lax.dot_general(oh, x_ref[:], (((0,), (0,)), ((), ())),
                         preferred_element_type=_f32)
    did = dom_ref[:]
    dids = lax.broadcasted_iota(jnp.int32, (1, 5), 1)
    doh = (did == dids).astype(_f32)
    dom = jnp.dot(doh, dtab_ref[:], preferred_element_type=_f32)
    gamma = jnp.dot(dom, fgw_ref[:], preferred_element_type=_f32) + fgb_ref[:]
    beta = jnp.dot(dom, fbw_ref[:], preferred_element_type=_f32) + fbb_ref[:]
    gf = gamma * gf + beta
    h = _silu(jnp.dot(gf, hw1_ref[:], preferred_element_type=_f32) + hb1_ref[:])
    h = _silu(jnp.dot(h, hw2_ref[:], preferred_element_type=_f32) + hb2_ref[:])
    out_ref[:] = jnp.dot(h, hw3_ref[:], preferred_element_type=_f32) + hb3_ref[:]


def _readout_call(x, batch2, dom2, dtab, fgw, fgb, fbw, fbb,
                  hw1, hb1, hw2, hb2, hw3, hb3):
    return pl.pallas_call(
        _readout_body,
        out_shape=jax.ShapeDtypeStruct((NGRAPH, 1), _f32),
    )(x, batch2, dom2, dtab, fgw, fgb, fbw, fbb, hw1, hb1, hw2, hb2, hw3, hb3)


def kernel(edge_attr, distances, atom_emb, domain_table, msg_W1, msg_b1,
           msg_W2, msg_b2, msg_W3, msg_b3, upd_W1, upd_b1, upd_W2, upd_b2,
           int_W1, int_b1, int_W2, int_b2, ln_g, ln_b, film_gW, film_gb,
           film_bW, film_bb, head_W1, head_b1, head_W2, head_b2, head_W3,
           head_b3, atomic_numbers, edge_index, batch, domain_ids):
    del edge_attr
    row = edge_index[0]
    col = edge_index[1]
    d2 = distances.reshape(N_EDGES, 1)
    an2 = atomic_numbers.reshape(N_NODES, 1)
    batch2 = batch.reshape(N_NODES, 1)
    dom2 = domain_ids.reshape(NGRAPH, 1)
    zeros = jnp.zeros((N_NODES, HID), _f32)

    a_w = [msg_W1[i, :HID] for i in range(NBLK)]
    b_w = [msg_W1[i, HID:2 * HID] for i in range(NBLK)]
    c_w = [msg_W1[i, 2 * HID:] for i in range(NBLK)]
    row2 = lambda v: v.reshape(1, -1)

    x, hr, hc = _prep_call(an2, atom_emb, a_w[0], b_w[0])

    for i in range(NBLK):
        hrg, hcg = _sc_gather(hr, hc, row, col)
        e3 = _edge_call(d2, hrg, hcg, c_w[i], row2(msg_b1[i]), msg_W2[i],
                        row2(msg_b2[i]), msg_W3[i], row2(msg_b3[i]))
        agg2 = _sc_scatter(e3, col, zeros)
        emit = i < NBLK - 1
        nxt = (i + 1) % NBLK
        outs = _node_call(emit, x, agg2[0], agg2[1],
                          upd_W1[i, :HID], upd_W1[i, HID:], row2(upd_b1[i]),
                          upd_W2[i], row2(upd_b2[i]),
                          int_W1[i], row2(int_b1[i]),
                          int_W2[i], row2(int_b2[i]),
                          row2(ln_g[i]), row2(ln_b[i]),
                          a_w[nxt], b_w[nxt])
        if emit:
            x, hr, hc = outs
        else:
            x = outs[0]

    out = _readout_call(x, batch2, dom2, domain_table, film_gW,
                        row2(film_gb), film_bW, row2(film_bb),
                        head_W1, row2(head_b1), head_W2, row2(head_b2),
                        head_W3, head_b3.reshape(1, 1))
    return out[:, 0]


# speedup vs baseline: 1.7864x; 1.7864x over previous
import functools

import jax
import jax.numpy as jnp
from jax import lax
from jax.experimental import pallas as pl
from jax.experimental.pallas import tpu as pltpu
from jax.experimental.pallas import tpu_sc as plsc

N_NODES = 10000
N_EDGES = 320000
HID = 128
NF = 128
NBLK = 2
RAD = 50
NGRAPH = 64
FILM = 16
CUT = 10.0

NC = 2
NS = 16
NW = NC * NS
EPW = N_EDGES // NW
CH = 80
NCH = EPW // CH
NPAD = 10240
RPT = NPAD // NS

_f32 = jnp.float32


def _silu(x):
    return x * jax.nn.sigmoid(x)


def _sc_gather_body(hr_hbm, hc_hbm, row_hbm, col_hbm, hrg_hbm, hcg_hbm,
                    idxr, idxc, buf_a, buf_b, sem_a, sem_b):
    cid = lax.axis_index("c")
    sid = lax.axis_index("s")
    wid = sid * NC + cid
    base0 = wid * EPW

    def chunk(j, carry):
        base = base0 + j * CH
        pltpu.sync_copy(row_hbm.at[pl.ds(base, CH)], idxr)
        pltpu.sync_copy(col_hbm.at[pl.ds(base, CH)], idxc)
        cp_a = pltpu.async_copy(hr_hbm.at[idxr], buf_a, sem_a)
        cp_b = pltpu.async_copy(hc_hbm.at[idxc], buf_b, sem_b)
        cp_a.wait()
        cp_b.wait()
        pltpu.sync_copy(buf_a, hrg_hbm.at[pl.ds(base, CH)])
        pltpu.sync_copy(buf_b, hcg_hbm.at[pl.ds(base, CH)])
        return carry

    lax.fori_loop(0, NCH, chunk, 0, unroll=False)


@functools.cache
def _sc_gather_kernel():
    return pl.kernel(
        _sc_gather_body,
        out_type=(
            jax.ShapeDtypeStruct((N_EDGES, HID), _f32),
            jax.ShapeDtypeStruct((N_EDGES, HID), _f32),
        ),
        mesh=plsc.VectorSubcoreMesh(core_axis_name="c", subcore_axis_name="s"),
        scratch_types=[
            pltpu.VMEM((CH,), jnp.int32),
            pltpu.VMEM((CH,), jnp.int32),
            pltpu.VMEM((CH, HID), _f32),
            pltpu.VMEM((CH, HID), _f32),
            pltpu.SemaphoreType.DMA,
            pltpu.SemaphoreType.DMA,
        ],
    )


def _sc_gather(hr, hc, row, col):
    return _sc_gather_kernel()(hr, hc, row, col)


def _sc_scatter_body(e_hbm, col_hbm, zeros_hbm, agg_hbm,
                     idxc, dbuf, acc, sem):
    del sem
    cid = lax.axis_index("c")
    sid = lax.axis_index("s")
    wid = sid * NC + cid
    base0 = wid * EPW

    pltpu.sync_copy(zeros_hbm.at[pl.ds(sid * RPT, RPT)],
                    acc.at[pl.ds(sid * RPT, RPT)])
    plsc.subcore_barrier()

    def chunk(j, carry):
        base = base0 + j * CH
        pltpu.sync_copy(col_hbm.at[pl.ds(base, CH)], idxc.at[0])
        pltpu.sync_copy(e_hbm.at[pl.ds(base, CH)], dbuf)
        pltpu.sync_copy(dbuf, acc.at[idxc.at[0]], add=True)
        return carry

    lax.fori_loop(0, NCH, chunk, 0, unroll=False)
    plsc.subcore_barrier()
    pltpu.sync_copy(acc.at[pl.ds(sid * RPT, RPT)],
                    agg_hbm.at[cid, pl.ds(sid * RPT, RPT)])


@functools.cache
def _sc_scatter_kernel():
    return pl.kernel(
        _sc_scatter_body,
        out_type=jax.ShapeDtypeStruct((NC, NPAD, HID), _f32),
        mesh=plsc.VectorSubcoreMesh(core_axis_name="c", subcore_axis_name="s"),
        scratch_types=[
            pltpu.VMEM((1, CH), jnp.int32),
            pltpu.VMEM((CH, HID), _f32),
            pltpu.VMEM_SHARED((NPAD, HID), _f32),
            pltpu.SemaphoreType.DMA,
        ],
    )


def _sc_scatter(e3, col, zeros):
    return _sc_scatter_kernel()(e3, col, zeros)


TN = 2000
TE = 2000


def _prep_body(an_ref, emb_ref, a_ref, b_ref, x_ref, hr_ref, hc_ref):
    an = an_ref[:]
    ids = lax.broadcasted_iota(jnp.int32, (1, 100), 1)
    oh = (an == ids).astype(_f32)
    x = jnp.dot(oh, emb_ref[:], preferred_element_type=_f32, precision=lax.Precision.HIGHEST)
    x_ref[:] = x
    hr_ref[:] = jnp.dot(x, a_ref[:], preferred_element_type=_f32, precision=lax.Precision.HIGHEST)
    hc_ref[:] = jnp.dot(x, b_ref[:], preferred_element_type=_f32, precision=lax.Precision.HIGHEST)


def _prep_call(an2, atom_emb, a_w, b_w):
    grid = N_NODES // TN
    return pl.pallas_call(
        _prep_body,
        grid=(grid,),
        in_specs=[
            pl.BlockSpec((TN, 1), lambda i: (i, 0)),
            pl.BlockSpec((100, HID), lambda i: (0, 0)),
            pl.BlockSpec((HID, HID), lambda i: (0, 0)),
            pl.BlockSpec((HID, HID), lambda i: (0, 0)),
        ],
        out_specs=[
            pl.BlockSpec((TN, HID), lambda i: (i, 0)),
            pl.BlockSpec((TN, HID), lambda i: (i, 0)),
            pl.BlockSpec((TN, HID), lambda i: (i, 0)),
        ],
        out_shape=[
            jax.ShapeDtypeStruct((N_NODES, HID), _f32),
            jax.ShapeDtypeStruct((N_NODES, HID), _f32),
            jax.ShapeDtypeStruct((N_NODES, HID), _f32),
        ],
    )(an2, atom_emb, a_w, b_w)


def _edge_body(d_ref, hrg_ref, hcg_ref, c_ref, b1_ref, w2_ref, b2_ref,
               w3_ref, b3_ref, out_ref):
    step = CUT / (RAD - 1)
    coeff = -0.5 / (step * step)
    off = lax.broadcasted_iota(jnp.int32, (1, RAD), 1).astype(_f32) * step
    d = d_ref[:]
    diff = d - off
    rbf = jnp.exp(coeff * diff * diff)
    pre = (hrg_ref[:] + hcg_ref[:] + b1_ref[:]
           + jnp.dot(rbf, c_ref[:], preferred_element_type=_f32, precision=lax.Precision.HIGHEST))
    e1 = _silu(pre)
    z = jnp.dot(e1, w2_ref[:], preferred_element_type=_f32, precision=lax.Precision.HIGHEST) + b2_ref[:]
    e2 = _silu(z)
    out_ref[:] = jnp.dot(e2, w3_ref[:], preferred_element_type=_f32, precision=lax.Precision.HIGHEST) + b3_ref[:]


def _edge_call(d2, hrg, hcg, c_w, b1, w2, b2, w3, b3):
    grid = N_EDGES // TE
    return pl.pallas_call(
        _edge_body,
        grid=(grid,),
        in_specs=[
            pl.BlockSpec((TE, 1), lambda i: (i, 0)),
            pl.BlockSpec((TE, HID), lambda i: (i, 0)),
            pl.BlockSpec((TE, HID), lambda i: (i, 0)),
            pl.BlockSpec((RAD, NF), lambda i: (0, 0)),
            pl.BlockSpec((1, NF), lambda i: (0, 0)),
            pl.BlockSpec((NF, NF), lambda i: (0, 0)),
            pl.BlockSpec((1, NF), lambda i: (0, 0)),
            pl.BlockSpec((NF, HID), lambda i: (0, 0)),
            pl.BlockSpec((1, HID), lambda i: (0, 0)),
        ],
        out_specs=pl.BlockSpec((TE, HID), lambda i: (i, 0)),
        out_shape=jax.ShapeDtypeStruct((N_EDGES, HID), _f32),
    )(d2, hrg, hcg, c_w, b1, w2, b2, w3, b3)


def _node_body(emit_proj, x_ref, a0_ref, a1_ref, up_ref, uq_ref, ub1_ref,
               uw2_ref, ub2_ref, iw1_ref, ib1_ref, iw2_ref, ib2_ref,
               g_ref, b_ref, an_ref, bn_ref, *out_refs):
    x = x_ref[:]
    agg = a0_ref[:] + a1_ref[:]
    u = (jnp.dot(x, up_ref[:], preferred_element_type=_f32, precision=lax.Precision.HIGHEST)
         + jnp.dot(agg, uq_ref[:], preferred_element_type=_f32, precision=lax.Precision.HIGHEST) + ub1_ref[:])
    u = jnp.dot(_silu(u), uw2_ref[:], preferred_element_type=_f32, precision=lax.Precision.HIGHEST) + ub2_ref[:]
    t = _silu(jnp.dot(u, iw1_ref[:], preferred_element_type=_f32, precision=lax.Precision.HIGHEST) + ib1_ref[:])
    v = jnp.dot(t, iw2_ref[:], preferred_element_type=_f32, precision=lax.Precision.HIGHEST) + ib2_ref[:]
    y = x + v
    m = jnp.mean(y, axis=-1, keepdims=True)
    yc = y - m
    var = jnp.mean(yc * yc, axis=-1, keepdims=True)
    xn = yc / jnp.sqrt(var + 1e-5) * g_ref[:] + b_ref[:]
    out_refs[0][:] = xn
    if emit_proj:
        out_refs[1][:] = jnp.dot(xn, an_ref[:], preferred_element_type=_f32, precision=lax.Precision.HIGHEST)
        out_refs[2][:] = jnp.dot(xn, bn_ref[:], preferred_element_type=_f32, precision=lax.Precision.HIGHEST)


def _node_call(emit_proj, x, a0, a1, up, uq, ub1, uw2, ub2, iw1, ib1, iw2,
               ib2, g, b, an_w, bn_w):
    grid = N_NODES // TN
    n_out = 3 if emit_proj else 1
    full = lambda shape: pl.BlockSpec(shape, lambda i: (0, 0))
    rows = pl.BlockSpec((TN, HID), lambda i: (i, 0))
    out = pl.pallas_call(
        functools.partial(_node_body, emit_proj),
        grid=(grid,),
        in_specs=[rows, rows, rows,
                  full((HID, HID)), full((HID, HID)), full((1, HID)),
                  full((HID, HID)), full((1, HID)),
                  full((HID, HID)), full((1, HID)),
                  full((HID, HID)), full((1, HID)),
                  full((1, HID)), full((1, HID)),
                  full((HID, HID)), full((HID, HID))],
        out_specs=[rows] * n_out,
        out_shape=[jax.ShapeDtypeStruct((N_NODES, HID), _f32)] * n_out,
    )(x, a0, a1, up, uq, ub1, uw2, ub2, iw1, ib1, iw2, ib2, g, b, an_w, bn_w)
    return out


def _readout_body(x_ref, batch_ref, dom_ref, dtab_ref, fgw_ref, fgb_ref,
                  fbw_ref, fbb_ref, hw1_ref, hb1_ref, hw2_ref, hb2_ref,
                  hw3_ref, hb3_ref, out_ref):
    bvec = batch_ref[:]
    gids = lax.broadcasted_iota(jnp.int32, (1, NGRAPH), 1)
    oh = (bvec == gids).astype(_f32)
    gf = lax.dot_general(oh, x_ref[:], (((0,), (0,)), ((), ())),
                         preferred_element_type=_f32, precision=lax.Precision.HIGHEST)
    did = dom_ref[:]
    dids = lax.broadcasted_iota(jnp.int32, (1, 5), 1)
    doh = (did == dids).astype(_f32)
    dom = jnp.dot(doh, dtab_ref[:], preferred_element_type=_f32, precision=lax.Precision.HIGHEST)
    gamma = jnp.dot(dom, fgw_ref[:], preferred_element_type=_f32, precision=lax.Precision.HIGHEST) + fgb_ref[:]
    beta = jnp.dot(dom, fbw_ref[:], preferred_element_type=_f32, precision=lax.Precision.HIGHEST) + fbb_ref[:]
    gf = gamma * gf + beta
    h = _silu(jnp.dot(gf, hw1_ref[:], preferred_element_type=_f32, precision=lax.Precision.HIGHEST) + hb1_ref[:])
    h = _silu(jnp.dot(h, hw2_ref[:], preferred_element_type=_f32, precision=lax.Precision.HIGHEST) + hb2_ref[:])
    out_ref[:] = jnp.dot(h, hw3_ref[:], preferred_element_type=_f32, precision=lax.Precision.HIGHEST) + hb3_ref[:]


def _readout_call(x, batch2, dom2, dtab, fgw, fgb, fbw, fbb,
                  hw1, hb1, hw2, hb2, hw3, hb3):
    return pl.pallas_call(
        _readout_body,
        out_shape=jax.ShapeDtypeStruct((NGRAPH, 1), _f32),
    )(x, batch2, dom2, dtab, fgw, fgb, fbw, fbb, hw1, hb1, hw2, hb2, hw3, hb3)


def kernel(edge_attr, distances, atom_emb, domain_table, msg_W1, msg_b1,
           msg_W2, msg_b2, msg_W3, msg_b3, upd_W1, upd_b1, upd_W2, upd_b2,
           int_W1, int_b1, int_W2, int_b2, ln_g, ln_b, film_gW, film_gb,
           film_bW, film_bb, head_W1, head_b1, head_W2, head_b2, head_W3,
           head_b3, atomic_numbers, edge_index, batch, domain_ids):
    del edge_attr
    row = edge_index[0]
    col = edge_index[1]
    d2 = distances.reshape(N_EDGES, 1)
    an2 = atomic_numbers.reshape(N_NODES, 1)
    batch2 = batch.reshape(N_NODES, 1)
    dom2 = domain_ids.reshape(NGRAPH, 1)
    zeros = jnp.zeros((NPAD, HID), _f32)

    a_w = [msg_W1[i, :HID] for i in range(NBLK)]
    b_w = [msg_W1[i, HID:2 * HID] for i in range(NBLK)]
    c_w = [msg_W1[i, 2 * HID:] for i in range(NBLK)]
    row2 = lambda v: v.reshape(1, -1)

    x, hr, hc = _prep_call(an2, atom_emb, a_w[0], b_w[0])

    for i in range(NBLK):
        hrg, hcg = _sc_gather(hr, hc, row, col)
        e3 = _edge_call(d2, hrg, hcg, c_w[i], row2(msg_b1[i]), msg_W2[i],
                        row2(msg_b2[i]), msg_W3[i], row2(msg_b3[i]))
        agg2 = _sc_scatter(e3, col, zeros)
        emit = i < NBLK - 1
        nxt = (i + 1) % NBLK
        outs = _node_call(emit, x, agg2[0, :N_NODES], agg2[1, :N_NODES],
                          upd_W1[i, :HID], upd_W1[i, HID:], row2(upd_b1[i]),
                          upd_W2[i], row2(upd_b2[i]),
                          int_W1[i], row2(int_b1[i]),
                          int_W2[i], row2(int_b2[i]),
                          row2(ln_g[i]), row2(ln_b[i]),
                          a_w[nxt], b_w[nxt])
        if emit:
            x, hr, hc = outs
        else:
            x = outs[0]

    out = _readout_call(x, batch2, dom2, domain_table, film_gW,
                        row2(film_gb), film_bW, row2(film_bb),
                        head_W1, row2(head_b1), head_W2, row2(head_b2),
                        head_W3, head_b3.reshape(1, 1))
    return out[:, 0]
